# baseline (speedup 1.0000x reference)
import jax
import jax.numpy as jnp
from jax.experimental import pallas as pl
from jax.experimental.pallas import tpu as pltpu
from jax.experimental.pallas import tpu_sc as plsc

_WINDOW = 128


def _sc_gather(table, indices):
    num_indices = indices.shape[0]
    value_dim = table.shape[1]
    idx2d = indices.reshape((1, num_indices))
    mesh = plsc.VectorSubcoreMesh(core_axis_name="core", subcore_axis_name="subcore")

    @pl.kernel(
        out_type=jax.ShapeDtypeStruct((num_indices, value_dim), table.dtype),
        mesh=mesh,
    )
    def kernel(x_hbm, i_hbm, o_hbm):
        def body(i_vmem, o_vmem):
            pltpu.sync_copy(x_hbm.at[i_vmem.at[0]], o_vmem)

        pltpu.emit_pipeline(
            body,
            grid=(num_indices // _WINDOW,),
            in_specs=[pl.BlockSpec((1, _WINDOW), index_map=lambda i: (0, i))],
            out_specs=[pl.BlockSpec((_WINDOW, value_dim), index_map=lambda i: (i, 0))],
            core_axis_name=("core", "subcore"),
            dimension_semantics=(pltpu.PARALLEL,),
        )(i_hbm, o_hbm)

    return kernel(table, idx2d)


def kernel(xyz1, xyz2, points2, W, b):
    B, _, N = xyz1.shape
    S = xyz2.shape[2]
    Cout, Cin = W.shape
    table = jnp.transpose(points2, (0, 2, 1)).reshape(B * S, Cin)
    n_idx = B * N * 3
    idx = (jnp.arange(n_idx, dtype=jnp.uint32) * jnp.uint32(2654435761)) % jnp.uint32(B * S)
    gathered = _sc_gather(table, idx.astype(jnp.int32))
    interp = gathered.reshape(B, N, 3, Cin).sum(axis=2)
    out = jnp.einsum("oi,bni->bon", W, interp) + b[None, :, None]
    return out.astype(jnp.float32)


# speedup vs baseline: 2.8009x; 2.8009x over previous
import jax
import jax.numpy as jnp
from jax.experimental import pallas as pl
from jax.experimental.pallas import tpu as pltpu


def _interp_kernel(x1_ref, x2_ref, p2_ref, w_ref, b_ref, out_ref):
    x1 = x1_ref[0]
    x2 = x2_ref[0]
    S = x2.shape[1]
    NT = x1.shape[1]

    mm = jax.lax.dot_general(x2, x1, (((0,), (0,)), ((), ())),
                             preferred_element_type=jnp.float32)
    s1 = (x1[0] * x1[0] + x1[1] * x1[1]) + x1[2] * x1[2]
    s2 = (x2[0] * x2[0] + x2[1] * x2[1]) + x2[2] * x2[2]
    d = -2.0 * mm
    d = d + s1[None, :]
    d = d + s2[:, None]

    iota = jax.lax.broadcasted_iota(jnp.int32, (S, NT), 0)
    mins, idxs = [], []
    for k in range(3):
        m = jnp.min(d, axis=0, keepdims=True)
        i = jnp.min(jnp.where(d == m, iota, S), axis=0, keepdims=True)
        mins.append(m)
        idxs.append(i)
        if k < 2:
            d = jnp.where(iota == i, jnp.float32(jnp.inf), d)

    recip = [1.0 / (m + 1e-8) for m in mins]
    norm = (recip[0] + recip[1]) + recip[2]

    wgt = [recip[k] / norm for k in range(3)]
    zero = jnp.zeros((S, NT), jnp.float32)
    a = jnp.where(
        iota == idxs[0], wgt[0],
        jnp.where(iota == idxs[1], wgt[1],
                  jnp.where(iota == idxs[2], wgt[2], zero)))

    interp = jnp.dot(p2_ref[0], a, preferred_element_type=jnp.float32)
    out = jnp.dot(w_ref[...], interp, preferred_element_type=jnp.float32)
    out_ref[0] = out + b_ref[:, 0:1]


def kernel(xyz1, xyz2, points2, W, b):
    B, _, N = xyz1.shape
    S = xyz2.shape[2]
    Cout, Cin = W.shape
    NT = 1024
    grid = (B, N // NT)
    return pl.pallas_call(
        _interp_kernel,
        grid=grid,
        in_specs=[
            pl.BlockSpec((1, 3, NT), lambda bb, nn: (bb, 0, nn)),
            pl.BlockSpec((1, 3, S), lambda bb, nn: (bb, 0, 0)),
            pl.BlockSpec((1, Cin, S), lambda bb, nn: (bb, 0, 0)),
            pl.BlockSpec((Cout, Cin), lambda bb, nn: (0, 0)),
            pl.BlockSpec((Cout, 128), lambda bb, nn: (0, 0)),
        ],
        out_specs=pl.BlockSpec((1, Cout, NT), lambda bb, nn: (bb, 0, nn)),
        out_shape=jax.ShapeDtypeStruct((B, Cout, N), jnp.float32),
        compiler_params=pltpu.CompilerParams(
            dimension_semantics=("parallel", "parallel"),
        ),
    )(xyz1, xyz2, points2, W, jnp.broadcast_to(b[:, None], (Cout, 128)))
